# baseline (speedup 1.0000x reference)
import jax
import jax.numpy as jnp
from jax.experimental import pallas as pl

HEADS = 4
HID = 16
OUT = 6
ENC = 16
PROTO = 64

G = 64


def _leaky(x):
    return jnp.where(x >= 0, x, 0.2 * x)


def _layer(h_ext, f_dim, maskfull, kron4, seg4, repg, eye4):
    h = h_ext[:, :f_dim]
    a_s = h_ext[:, f_dim:f_dim + HEADS]
    a_d = h_ext[:, f_dim + HEADS:f_dim + 2 * HEADS]
    asT = jax.lax.dot_general(
        eye4, a_s, (((1,), (1,)), ((), ())),
        preferred_element_type=jnp.float32)
    rows = []
    for c in range(G):
        blocks = [jnp.broadcast_to(asT[hd:hd + 1, c * 64:(c + 1) * 64],
                                   (64, 64)) for hd in range(HEADS)]
        rows.append(jnp.concatenate(blocks, axis=1))
    src = jnp.concatenate(rows, axis=0)
    dst = jnp.dot(a_d, kron4, preferred_element_type=jnp.float32)
    s = _leaky(dst + src)
    asmax = jnp.max(a_s.reshape(G, 64, HEADS), axis=1)
    m = _leaky(a_d + jnp.dot(repg, asmax,
                             preferred_element_type=jnp.float32))
    p = jnp.exp(s - jnp.dot(m, kron4, preferred_element_type=jnp.float32))
    den = jnp.dot(p, seg4, preferred_element_type=jnp.float32)
    attn = p * jnp.dot(1.0 / den, kron4,
                       preferred_element_type=jnp.float32)
    outs = []
    for c in range(G):
        hc = h[c * 64:(c + 1) * 64, :]
        bc = jnp.concatenate([hc, hc, hc, hc], axis=0) * maskfull
        outs.append(jnp.dot(attn[c * 64:(c + 1) * 64, :], bc,
                            preferred_element_type=jnp.float32))
    return jnp.concatenate(outs, axis=0)


def _gnn_kernel(xsT_ref, pos_ref, w0r0_ref, w0rest_ref, w1_ref, w2_ref,
                b0_ref, b1_ref, b2_ref,
                mask16_ref, mask6_ref, eye4_ref, kron4_ref, seg4_ref,
                repg_ref,
                linw_ref, linb_ref, mmean_ref, out_ref):
    eye4 = eye4_ref[...]
    kron4 = kron4_ref[...]
    seg4 = seg4_ref[...]
    repg = repg_ref[...]
    hpos = jnp.dot(pos_ref[0], w0rest_ref[...],
                   preferred_element_type=jnp.float32)
    xsT = xsT_ref[0]
    h_parts = []
    for c in range(G):
        h_parts.append(xsT[:, c:c + 1] * w0r0_ref[...] + hpos)
    h0 = jnp.concatenate(h_parts, axis=0)
    x1 = _layer(h0, HEADS * HID, mask16_ref[...], kron4, seg4, repg,
                eye4) + b0_ref[...]

    h1 = jnp.dot(x1, w1_ref[...], preferred_element_type=jnp.float32)
    x2 = _layer(h1, HEADS * HID, mask16_ref[...], kron4, seg4, repg,
                eye4) + b1_ref[...]

    h2 = jnp.dot(x2, w2_ref[...], preferred_element_type=jnp.float32)
    x3 = _layer(h2, HEADS * OUT, mask6_ref[...], kron4, seg4, repg,
                eye4) + b2_ref[...]

    y = jnp.dot(x3, linw_ref[...], preferred_element_type=jnp.float32) \
        + linb_ref[...]
    out_ref[...] = jnp.dot(mmean_ref[...], y,
                           preferred_element_type=jnp.float32)


@jax.jit
def kernel(batch_xs, batch_pos_enc, W0, a_src0, a_dst0, b0,
           W1, a_src1, a_dst1, b1, W2, a_src2, a_dst2, b2, linW, linb):
    bs, n_row, n_col = batch_xs.shape
    ncliq = bs * n_row

    xsT = batch_xs.reshape(ncliq // G, G, n_col).transpose(0, 2, 1)

    def amat(a, d):
        return jnp.kron(jnp.eye(HEADS, dtype=jnp.float32),
                        jnp.ones((d, 1), jnp.float32)) * a.reshape(-1, 1)

    def wext(w, a_src, a_dst, d):
        return jnp.concatenate(
            [w, w @ amat(a_src, d), w @ amat(a_dst, d)], axis=1)

    W0e = wext(W0, a_src0, a_dst0, HID)
    W1e = wext(W1, a_src1, a_dst1, HID)
    W2e = wext(W2, a_src2, a_dst2, OUT)

    mask16 = jnp.kron(jnp.kron(jnp.eye(HEADS, dtype=jnp.float32),
                               jnp.ones((1, HID), jnp.float32)),
                      jnp.ones((n_col, 1), jnp.float32))
    mask6 = jnp.kron(jnp.kron(jnp.eye(HEADS, dtype=jnp.float32),
                              jnp.ones((1, OUT), jnp.float32)),
                     jnp.ones((n_col, 1), jnp.float32))
    eye4 = jnp.eye(HEADS, dtype=jnp.float32)
    kron4 = jnp.kron(jnp.eye(HEADS, dtype=jnp.float32),
                     jnp.ones((1, n_col), jnp.float32))
    seg4 = jnp.kron(jnp.eye(HEADS, dtype=jnp.float32),
                    jnp.ones((n_col, 1), jnp.float32))
    repg = jnp.kron(jnp.eye(G, dtype=jnp.float32),
                    jnp.ones((n_col, 1), jnp.float32))
    mmean = jnp.kron(jnp.eye(G, dtype=jnp.float32),
                     jnp.full((1, n_col), 1.0 / n_col, jnp.float32))

    grid = (ncliq // G,)
    rep = lambda *shape: pl.BlockSpec(shape, lambda i: (0,) * len(shape))
    out = pl.pallas_call(
        _gnn_kernel,
        grid=grid,
        in_specs=[
            pl.BlockSpec((1, n_col, G), lambda i: (i, 0, 0)),
            pl.BlockSpec((1, n_col, ENC), lambda i: (i // (n_row // G), 0, 0)),
            rep(1, 2 * HEADS + HEADS * HID),
            rep(ENC, 2 * HEADS + HEADS * HID),
            rep(HEADS * HID, 2 * HEADS + HEADS * HID),
            rep(HEADS * HID, 2 * HEADS + HEADS * OUT),
            rep(1, HEADS * HID), rep(1, HEADS * HID),
            rep(1, HEADS * OUT),
            rep(HEADS * n_col, HEADS * HID),
            rep(HEADS * n_col, HEADS * OUT),
            rep(HEADS, HEADS),
            rep(HEADS, HEADS * n_col),
            rep(HEADS * n_col, HEADS),
            rep(G * n_col, G),
            rep(HEADS * OUT, PROTO), rep(1, PROTO),
            rep(G, G * n_col),
        ],
        out_specs=pl.BlockSpec((G, PROTO), lambda i: (i, 0)),
        out_shape=jax.ShapeDtypeStruct((ncliq, PROTO), jnp.float32),
    )(xsT, batch_pos_enc, W0e[0:1, :], W0e[1:, :], W1e, W2e,
      b0.reshape(1, -1), b1.reshape(1, -1), b2.reshape(1, -1),
      mask16, mask6, eye4, kron4, seg4, repg, linW, linb.reshape(1, -1),
      mmean)

    return out.reshape(bs, n_row, PROTO)


# speedup vs baseline: 1.0697x; 1.0697x over previous
import jax
import jax.numpy as jnp
from jax.experimental import pallas as pl

HEADS = 4
HID = 16
OUT = 6
ENC = 16
PROTO = 64

G = 32


def _leaky(x):
    return jnp.maximum(x, 0.2 * x)


def _layer(h, a_s, a_d, dst, f_dim, maskfull, kronf, kron4, seg4, repg,
           eye4):
    asT = jax.lax.dot_general(
        eye4, a_s, (((1,), (1,)), ((), ())),
        preferred_element_type=jnp.float32)
    rows = []
    for c in range(G):
        blocks = [jnp.broadcast_to(asT[hd:hd + 1, c * 64:(c + 1) * 64],
                                   (64, 64)) for hd in range(HEADS)]
        rows.append(jnp.concatenate(blocks, axis=1))
    src = jnp.concatenate(rows, axis=0)
    s = _leaky(dst + src)
    asmax = jnp.max(a_s.reshape(G, 64, HEADS), axis=1)
    m = _leaky(a_d + jnp.dot(repg, asmax,
                             preferred_element_type=jnp.float32))
    p = jnp.exp(s - jnp.dot(m, kron4, preferred_element_type=jnp.float32))
    den = jnp.dot(p, seg4, preferred_element_type=jnp.float32)
    outs = []
    for c in range(G):
        hc = h[c * 64:(c + 1) * 64, :]
        bc = jnp.concatenate([hc, hc, hc, hc], axis=0) * maskfull
        outs.append(jnp.dot(p[c * 64:(c + 1) * 64, :], bc,
                            preferred_element_type=jnp.float32))
    out = jnp.concatenate(outs, axis=0)
    return out * jnp.dot(1.0 / den, kronf,
                         preferred_element_type=jnp.float32)


def _gnn_kernel(xsT_ref, pos_ref, w0r0_ref, w0rest_ref, w1_ref, w2_ref,
                b0_ref, b1_ref, b2_ref,
                mask16_ref, mask6_ref, eye4_ref, kronf16_ref, kronf6_ref,
                kron4_ref, seg4_ref, repg_ref,
                linw_ref, linb_ref, mmean_ref, out_ref):
    eye4 = eye4_ref[...]
    kron4 = kron4_ref[...]
    seg4 = seg4_ref[...]
    repg = repg_ref[...]
    hpos = jnp.dot(pos_ref[0], w0rest_ref[...],
                   preferred_element_type=jnp.float32)
    xsT = xsT_ref[0]
    F16 = HEADS * HID
    F6 = HEADS * OUT
    w0r0 = w0r0_ref[...]
    w0r0_dst = jnp.dot(w0r0[:, F16 + HEADS:F16 + 2 * HEADS], kron4,
                       preferred_element_type=jnp.float32)
    hpos_dst = jnp.dot(hpos[:, F16 + HEADS:F16 + 2 * HEADS], kron4,
                       preferred_element_type=jnp.float32)
    h_parts = []
    dst_parts = []
    for c in range(G):
        xcol = xsT[:, c:c + 1]
        h_parts.append(xcol * w0r0 + hpos)
        dst_parts.append(xcol * w0r0_dst + hpos_dst)
    h0 = jnp.concatenate(h_parts, axis=0)
    dst0 = jnp.concatenate(dst_parts, axis=0)
    x1 = _layer(h0[:, :F16], h0[:, F16:F16 + HEADS],
                h0[:, F16 + HEADS:F16 + 2 * HEADS],
                dst0, F16, mask16_ref[...],
                kronf16_ref[...], kron4, seg4, repg, eye4) + b0_ref[...]
    h1 = jnp.dot(x1, w1_ref[...], preferred_element_type=jnp.float32)
    x2 = _layer(h1[:, :F16], h1[:, F16:F16 + HEADS],
                h1[:, F16 + HEADS:F16 + 2 * HEADS],
                h1[:, F16 + 2 * HEADS:], F16, mask16_ref[...],
                kronf16_ref[...], kron4, seg4, repg, eye4) + b1_ref[...]

    h2 = jnp.dot(x2, w2_ref[...], preferred_element_type=jnp.float32)
    x3 = _layer(h2[:, :F6], h2[:, F6:F6 + HEADS],
                h2[:, F6 + HEADS:F6 + 2 * HEADS],
                h2[:, F6 + 2 * HEADS:], F6, mask6_ref[...],
                kronf6_ref[...], kron4, seg4, repg, eye4) + b2_ref[...]

    y = jnp.dot(x3, linw_ref[...], preferred_element_type=jnp.float32) \
        + linb_ref[...]
    out_ref[...] = jnp.dot(mmean_ref[...], y,
                           preferred_element_type=jnp.float32)


@jax.jit
def kernel(batch_xs, batch_pos_enc, W0, a_src0, a_dst0, b0,
           W1, a_src1, a_dst1, b1, W2, a_src2, a_dst2, b2, linW, linb):
    bs, n_row, n_col = batch_xs.shape
    ncliq = bs * n_row

    xsT = batch_xs.reshape(ncliq // G, G, n_col).transpose(0, 2, 1)

    def amat(a, d):
        return jnp.kron(jnp.eye(HEADS, dtype=jnp.float32),
                        jnp.ones((d, 1), jnp.float32)) * a.reshape(-1, 1)

    kron4c = jnp.kron(jnp.eye(HEADS, dtype=jnp.float32),
                      jnp.ones((1, n_col), jnp.float32))

    def wext(w, a_src, a_dst, d, with_dst_exp):
        cols = [w, w @ amat(a_src, d), w @ amat(a_dst, d)]
        if with_dst_exp:
            cols.append(w @ amat(a_dst, d) @ kron4c)
        return jnp.concatenate(cols, axis=1)

    W0e = wext(W0, a_src0, a_dst0, HID, False)
    W1e = wext(W1, a_src1, a_dst1, HID, True)
    W2e = wext(W2, a_src2, a_dst2, OUT, True)

    mask16 = jnp.kron(jnp.kron(jnp.eye(HEADS, dtype=jnp.float32),
                               jnp.ones((1, HID), jnp.float32)),
                      jnp.ones((n_col, 1), jnp.float32))
    mask6 = jnp.kron(jnp.kron(jnp.eye(HEADS, dtype=jnp.float32),
                              jnp.ones((1, OUT), jnp.float32)),
                     jnp.ones((n_col, 1), jnp.float32))
    eye4 = jnp.eye(HEADS, dtype=jnp.float32)
    kronf16 = jnp.kron(jnp.eye(HEADS, dtype=jnp.float32),
                       jnp.ones((1, HID), jnp.float32))
    kronf6 = jnp.kron(jnp.eye(HEADS, dtype=jnp.float32),
                      jnp.ones((1, OUT), jnp.float32))
    kron4 = kron4c
    seg4 = jnp.kron(jnp.eye(HEADS, dtype=jnp.float32),
                    jnp.ones((n_col, 1), jnp.float32))
    repg = jnp.kron(jnp.eye(G, dtype=jnp.float32),
                    jnp.ones((n_col, 1), jnp.float32))
    mmean = jnp.kron(jnp.eye(G, dtype=jnp.float32),
                     jnp.full((1, n_col), 1.0 / n_col, jnp.float32))

    grid = (ncliq // G,)
    rep = lambda *shape: pl.BlockSpec(shape, lambda i: (0,) * len(shape))
    out = pl.pallas_call(
        _gnn_kernel,
        grid=grid,
        in_specs=[
            pl.BlockSpec((1, n_col, G), lambda i: (i, 0, 0)),
            pl.BlockSpec((1, n_col, ENC), lambda i: (i // (n_row // G), 0, 0)),
            rep(1, 2 * HEADS + HEADS * HID),
            rep(ENC, 2 * HEADS + HEADS * HID),
            rep(HEADS * HID, 2 * HEADS + HEADS * HID + 256),
            rep(HEADS * HID, 2 * HEADS + HEADS * OUT + 256),
            rep(1, HEADS * HID), rep(1, HEADS * HID),
            rep(1, HEADS * OUT),
            rep(HEADS * n_col, HEADS * HID),
            rep(HEADS * n_col, HEADS * OUT),
            rep(HEADS, HEADS),
            rep(HEADS, HEADS * HID),
            rep(HEADS, HEADS * OUT),
            rep(HEADS, HEADS * n_col),
            rep(HEADS * n_col, HEADS),
            rep(G * n_col, G),
            rep(HEADS * OUT, PROTO), rep(1, PROTO),
            rep(G, G * n_col),
        ],
        out_specs=pl.BlockSpec((G, PROTO), lambda i: (i, 0)),
        out_shape=jax.ShapeDtypeStruct((ncliq, PROTO), jnp.float32),
    )(xsT, batch_pos_enc, W0e[0:1, :], W0e[1:, :], W1e, W2e,
      b0.reshape(1, -1), b1.reshape(1, -1), b2.reshape(1, -1),
      mask16, mask6, eye4, kronf16, kronf6, kron4, seg4, repg, linW,
      linb.reshape(1, -1), mmean)

    return out.reshape(bs, n_row, PROTO)


# speedup vs baseline: 1.2789x; 1.1956x over previous
import jax
import jax.numpy as jnp
from jax.experimental import pallas as pl

HEADS = 4
HID = 16
OUT = 6
ENC = 16
PROTO = 64

G = 32


def _leaky(x):
    return jnp.maximum(x, 0.2 * x)


def _layer(h, a_s, a_d, f_dim, maskfull, kronf, kron4, seg4, repg,
           eye4):
    asT = jax.lax.dot_general(
        eye4, a_s, (((1,), (1,)), ((), ())),
        preferred_element_type=jnp.float32)
    rows = []
    for c in range(G):
        blocks = [jnp.broadcast_to(asT[hd:hd + 1, c * 64:(c + 1) * 64],
                                   (64, 64)) for hd in range(HEADS)]
        rows.append(jnp.concatenate(blocks, axis=1))
    src = jnp.concatenate(rows, axis=0)
    dst = jnp.dot(a_d, kron4, preferred_element_type=jnp.float32)
    s = _leaky(dst + src)
    asmax = jnp.max(a_s.reshape(G, 64, HEADS), axis=1)
    m = _leaky(a_d + jnp.dot(repg, asmax,
                             preferred_element_type=jnp.float32))
    p = jnp.exp(s - jnp.dot(m, kron4, preferred_element_type=jnp.float32))
    den = jnp.dot(p, seg4, preferred_element_type=jnp.float32)
    outs = []
    for c in range(G):
        hc = h[c * 64:(c + 1) * 64, :]
        bc = jnp.concatenate([hc, hc, hc, hc], axis=0) * maskfull
        outs.append(jnp.dot(p[c * 64:(c + 1) * 64, :], bc,
                            preferred_element_type=jnp.float32))
    out = jnp.concatenate(outs, axis=0)
    return out * jnp.dot(1.0 / den, kronf,
                         preferred_element_type=jnp.float32)


def _gnn_kernel(xsT_ref, pos_ref, w0r0_ref, w0rest_ref, w1_ref, w2_ref,
                b0_ref, b1_ref, b2_ref,
                mask16_ref, mask6_ref, eye4_ref, kronf16_ref, kronf6_ref,
                kron4_ref, seg4_ref, repg_ref,
                linw_ref, linb_ref, mmean_ref, out_ref):
    eye4 = eye4_ref[...]
    kron4 = kron4_ref[...]
    seg4 = seg4_ref[...]
    repg = repg_ref[...]
    hpos = jnp.dot(pos_ref[0], w0rest_ref[...],
                   preferred_element_type=jnp.float32)
    xsT = xsT_ref[0]
    F16 = HEADS * HID
    F6 = HEADS * OUT
    w0r0 = w0r0_ref[...]
    h_parts = []
    for c in range(G):
        h_parts.append(xsT[:, c:c + 1] * w0r0 + hpos)
    h0 = jnp.concatenate(h_parts, axis=0)
    x1 = _layer(h0[:, :F16], h0[:, F16:F16 + HEADS],
                h0[:, F16 + HEADS:F16 + 2 * HEADS], F16, mask16_ref[...],
                kronf16_ref[...], kron4, seg4, repg, eye4) + b0_ref[...]

    h1 = jnp.dot(x1, w1_ref[...], preferred_element_type=jnp.float32)
    x2 = _layer(h1[:, :F16], h1[:, F16:F16 + HEADS],
                h1[:, F16 + HEADS:F16 + 2 * HEADS], F16, mask16_ref[...],
                kronf16_ref[...], kron4, seg4, repg, eye4) + b1_ref[...]

    h2 = jnp.dot(x2, w2_ref[...], preferred_element_type=jnp.float32)
    x3 = _layer(h2[:, :F6], h2[:, F6:F6 + HEADS],
                h2[:, F6 + HEADS:F6 + 2 * HEADS], F6, mask6_ref[...],
                kronf6_ref[...], kron4, seg4, repg, eye4) + b2_ref[...]

    y = jnp.dot(x3, linw_ref[...], preferred_element_type=jnp.float32) \
        + linb_ref[...]
    out_ref[...] = jnp.dot(mmean_ref[...], y,
                           preferred_element_type=jnp.float32)


@jax.jit
def kernel(batch_xs, batch_pos_enc, W0, a_src0, a_dst0, b0,
           W1, a_src1, a_dst1, b1, W2, a_src2, a_dst2, b2, linW, linb):
    bs, n_row, n_col = batch_xs.shape
    ncliq = bs * n_row

    xsT = batch_xs.reshape(ncliq // G, G, n_col).transpose(0, 2, 1)

    def amat(a, d):
        return jnp.kron(jnp.eye(HEADS, dtype=jnp.float32),
                        jnp.ones((d, 1), jnp.float32)) * a.reshape(-1, 1)

    kron4c = jnp.kron(jnp.eye(HEADS, dtype=jnp.float32),
                      jnp.ones((1, n_col), jnp.float32))

    def wext(w, a_src, a_dst, d):
        return jnp.concatenate(
            [w, w @ amat(a_src, d), w @ amat(a_dst, d)], axis=1)

    W0e = wext(W0, a_src0, a_dst0, HID)
    W1e = wext(W1, a_src1, a_dst1, HID)
    W2e = wext(W2, a_src2, a_dst2, OUT)

    mask16 = jnp.kron(jnp.kron(jnp.eye(HEADS, dtype=jnp.float32),
                               jnp.ones((1, HID), jnp.float32)),
                      jnp.ones((n_col, 1), jnp.float32))
    mask6 = jnp.kron(jnp.kron(jnp.eye(HEADS, dtype=jnp.float32),
                              jnp.ones((1, OUT), jnp.float32)),
                     jnp.ones((n_col, 1), jnp.float32))
    eye4 = jnp.eye(HEADS, dtype=jnp.float32)
    kronf16 = jnp.kron(jnp.eye(HEADS, dtype=jnp.float32),
                       jnp.ones((1, HID), jnp.float32))
    kronf6 = jnp.kron(jnp.eye(HEADS, dtype=jnp.float32),
                      jnp.ones((1, OUT), jnp.float32))
    kron4 = kron4c
    seg4 = jnp.kron(jnp.eye(HEADS, dtype=jnp.float32),
                    jnp.ones((n_col, 1), jnp.float32))
    repg = jnp.kron(jnp.eye(G, dtype=jnp.float32),
                    jnp.ones((n_col, 1), jnp.float32))
    mmean = jnp.kron(jnp.eye(G, dtype=jnp.float32),
                     jnp.full((1, n_col), 1.0 / n_col, jnp.float32))

    grid = (ncliq // G,)
    rep = lambda *shape: pl.BlockSpec(shape, lambda i: (0,) * len(shape))
    out = pl.pallas_call(
        _gnn_kernel,
        grid=grid,
        in_specs=[
            pl.BlockSpec((1, n_col, G), lambda i: (i, 0, 0)),
            pl.BlockSpec((1, n_col, ENC), lambda i: (i // (n_row // G), 0, 0)),
            rep(1, 2 * HEADS + HEADS * HID),
            rep(ENC, 2 * HEADS + HEADS * HID),
            rep(HEADS * HID, 2 * HEADS + HEADS * HID),
            rep(HEADS * HID, 2 * HEADS + HEADS * OUT),
            rep(1, HEADS * HID), rep(1, HEADS * HID),
            rep(1, HEADS * OUT),
            rep(HEADS * n_col, HEADS * HID),
            rep(HEADS * n_col, HEADS * OUT),
            rep(HEADS, HEADS),
            rep(HEADS, HEADS * HID),
            rep(HEADS, HEADS * OUT),
            rep(HEADS, HEADS * n_col),
            rep(HEADS * n_col, HEADS),
            rep(G * n_col, G),
            rep(HEADS * OUT, PROTO), rep(1, PROTO),
            rep(G, G * n_col),
        ],
        out_specs=pl.BlockSpec((G, PROTO), lambda i: (i, 0)),
        out_shape=jax.ShapeDtypeStruct((ncliq, PROTO), jnp.float32),
    )(xsT, batch_pos_enc, W0e[0:1, :], W0e[1:, :], W1e, W2e,
      b0.reshape(1, -1), b1.reshape(1, -1), b2.reshape(1, -1),
      mask16, mask6, eye4, kronf16, kronf6, kron4, seg4, repg, linW,
      linb.reshape(1, -1), mmean)

    return out.reshape(bs, n_row, PROTO)
